# Initial kernel scaffold; baseline (speedup 1.0000x reference)
import jax
import jax.numpy as jnp
from jax.experimental import pallas as pl


def kernel(target_input, context, neg, target_w, context_w):
    raise NotImplementedError("write your pallas kernel here")


# speedup vs baseline: 4.9772x; 4.9772x over previous
import functools

import jax
import jax.numpy as jnp
from jax import lax
from jax.experimental import pallas as pl
from jax.experimental.pallas import tpu as pltpu
from jax.experimental.pallas import tpu_sc as plsc

NC = 2
NS = 16
NW = NC * NS
IDXW = 128


def _sc_gather(target_input, context, ni_flat, sidx2d, target_w, context_w):
    B = target_input.shape[0]
    V, D = target_w.shape
    NEG = ni_flat.size // B
    BPW = B // NW
    NPC = BPW // IDXW
    NNP = BPW * NEG // IDXW
    B_SC = B // NC
    f32 = jnp.float32

    mesh = plsc.VectorSubcoreMesh(core_axis_name="c", subcore_axis_name="s")
    out_types = (jax.ShapeDtypeStruct((B, D), f32),) * 3

    @functools.partial(
        pl.kernel,
        mesh=mesh,
        out_type=out_types,
        compiler_params=pltpu.CompilerParams(use_tc_tiling_on_sc=False),
        scratch_types=[
            pltpu.VMEM((BPW,), jnp.int32),
            pltpu.VMEM((BPW,), jnp.int32),
            pltpu.VMEM((BPW * NEG,), jnp.int32),
            pltpu.VMEM((NNP, IDXW), jnp.int32),
            pltpu.VMEM((IDXW, D), f32),
            pltpu.VMEM((IDXW, D), f32),
            pltpu.VMEM((IDXW, D), f32),
            pltpu.VMEM((IDXW, D), f32),
            pltpu.VMEM_SHARED((B_SC, D), f32),
        ],
    )
    def sc_part(
        ti_hbm, ci_hbm, ni_hbm, si_hbm, tw_hbm, cw_hbm,
        v_hbm, u_hbm, ns_hbm,
        ti_v, ci_v, ni_v, si_v, vrows, urows, nrows, zbuf, sh_ns,
    ):
        s = lax.axis_index("s")
        c = lax.axis_index("c")
        wid = s * NC + c
        base = wid * BPW

        @pl.loop(0, IDXW)
        def _(r):
            for ch in range(D // 16):
                zbuf[r, pl.ds(ch * 16, 16)] = jnp.zeros((16,), f32)

        @pl.loop(0, NPC)
        def _(p):
            pltpu.sync_copy(zbuf, sh_ns.at[pl.ds(s * BPW + p * IDXW, IDXW)])

        pltpu.sync_copy(ti_hbm.at[pl.ds(base, BPW)], ti_v)
        pltpu.sync_copy(ci_hbm.at[pl.ds(base, BPW)], ci_v)
        pltpu.sync_copy(ni_hbm.at[pl.ds(base * NEG, BPW * NEG)], ni_v)
        pltpu.sync_copy(si_hbm.at[pl.ds(wid * NNP, NNP)], si_v)

        @pl.loop(0, NPC)
        def _(j):
            pltpu.sync_copy(tw_hbm.at[ti_v.at[pl.ds(j * IDXW, IDXW)]], vrows)
            pltpu.sync_copy(vrows, v_hbm.at[pl.ds(base + j * IDXW, IDXW)])
            pltpu.sync_copy(cw_hbm.at[ci_v.at[pl.ds(j * IDXW, IDXW)]], urows)
            pltpu.sync_copy(urows, u_hbm.at[pl.ds(base + j * IDXW, IDXW)])

        @pl.loop(0, NNP)
        def _(j):
            pltpu.sync_copy(cw_hbm.at[ni_v.at[pl.ds(j * IDXW, IDXW)]], nrows)
            pltpu.sync_copy(nrows, sh_ns.at[si_v.at[j]], add=True)

        @pl.loop(0, NPC)
        def _(p):
            pltpu.sync_copy(
                sh_ns.at[pl.ds(s * BPW + p * IDXW, IDXW)],
                ns_hbm.at[pl.ds(base + p * IDXW, IDXW)],
            )

    return sc_part(target_input, context, ni_flat, sidx2d, target_w, context_w)


def _tc_loss(v, u, ns):
    B, D = v.shape
    TB = 2048

    def body(v_ref, u_ref, ns_ref, o_ref):
        vv = v_ref[...]
        sp = jnp.sum(u_ref[...] * vv, axis=1, keepdims=True)
        sn = jnp.sum(ns_ref[...] * vv, axis=1, keepdims=True)
        ls = jax.nn.log_sigmoid(sp) + jax.nn.log_sigmoid(-sn)
        part = jnp.full((1, 1), -jnp.sum(ls) / B, jnp.float32)

        @pl.when(pl.program_id(0) == 0)
        def _():
            o_ref[...] = jnp.zeros((1, 1), jnp.float32)

        o_ref[...] += part

    out = pl.pallas_call(
        body,
        grid=(B // TB,),
        in_specs=[pl.BlockSpec((TB, D), lambda i: (i, 0))] * 3,
        out_specs=pl.BlockSpec((1, 1), lambda i: (0, 0)),
        out_shape=jax.ShapeDtypeStruct((1, 1), jnp.float32),
    )(v, u, ns)
    return out[0, 0]


def kernel(target_input, context, neg, target_w, context_w):
    B, NEG = neg.shape
    BPW = B // NW

    ni_flat = neg.reshape(-1)
    b = jnp.arange(B, dtype=jnp.int32)
    lidx = ((b // BPW) // NC) * BPW + (b % BPW)
    sidx2d = jnp.repeat(lidx, NEG).reshape(B * NEG // IDXW, IDXW)

    v, u, ns = _sc_gather(target_input, context, ni_flat, sidx2d, target_w, context_w)
    return _tc_loss(v, u, ns)


# speedup vs baseline: 5.1435x; 1.0334x over previous
import functools

import jax
import jax.numpy as jnp
from jax import lax
from jax.experimental import pallas as pl
from jax.experimental.pallas import tpu as pltpu
from jax.experimental.pallas import tpu_sc as plsc

NC = 2
NS = 16
NW = NC * NS
IDXW = 128
NBUF = 5


def _sc_gather(target_input, context, ni_flat, sidx2d, target_w, context_w):
    B = target_input.shape[0]
    V, D = target_w.shape
    NEG = ni_flat.size // B
    BPW = B // NW
    NPC = BPW // IDXW
    NNP = BPW * NEG // IDXW
    B_SC = B // NC
    f32 = jnp.float32

    mesh = plsc.VectorSubcoreMesh(core_axis_name="c", subcore_axis_name="s")
    out_types = (jax.ShapeDtypeStruct((B, D), f32),) * 3

    @functools.partial(
        pl.kernel,
        mesh=mesh,
        out_type=out_types,
        compiler_params=pltpu.CompilerParams(use_tc_tiling_on_sc=False),
        scratch_types=[
            pltpu.VMEM((BPW,), jnp.int32),
            pltpu.VMEM((BPW,), jnp.int32),
            pltpu.VMEM((BPW * NEG,), jnp.int32),
            pltpu.VMEM((NNP, IDXW), jnp.int32),
            pltpu.VMEM((IDXW, D), f32),
            pltpu.VMEM((IDXW, D), f32),
            pltpu.VMEM((NBUF * IDXW, D), f32),
            pltpu.VMEM((IDXW, D), f32),
            pltpu.VMEM_SHARED((B_SC, D), f32),
        ]
        + [pltpu.SemaphoreType.DMA] * (2 * NBUF),
    )
    def sc_part(
        ti_hbm, ci_hbm, ni_hbm, si_hbm, tw_hbm, cw_hbm,
        v_hbm, u_hbm, ns_hbm,
        ti_v, ci_v, ni_v, si_v, vrows, urows, nrows, zbuf, sh_ns, *sems,
    ):
        gsem, ssem = sems[:NBUF], sems[NBUF:]
        s = lax.axis_index("s")
        c = lax.axis_index("c")
        wid = s * NC + c
        base = wid * BPW

        @pl.loop(0, IDXW)
        def _(r):
            for ch in range(D // 16):
                zbuf[r, pl.ds(ch * 16, 16)] = jnp.zeros((16,), f32)

        @pl.loop(0, NPC)
        def _(p):
            pltpu.sync_copy(zbuf, sh_ns.at[pl.ds(s * BPW + p * IDXW, IDXW)])

        pltpu.sync_copy(ti_hbm.at[pl.ds(base, BPW)], ti_v)
        pltpu.sync_copy(ci_hbm.at[pl.ds(base, BPW)], ci_v)
        pltpu.sync_copy(ni_hbm.at[pl.ds(base * NEG, BPW * NEG)], ni_v)
        pltpu.sync_copy(si_hbm.at[pl.ds(wid * NNP, NNP)], si_v)

        @pl.loop(0, NPC)
        def _(j):
            pltpu.sync_copy(tw_hbm.at[ti_v.at[pl.ds(j * IDXW, IDXW)]], vrows)
            pltpu.sync_copy(vrows, v_hbm.at[pl.ds(base + j * IDXW, IDXW)])
            pltpu.sync_copy(cw_hbm.at[ci_v.at[pl.ds(j * IDXW, IDXW)]], urows)
            pltpu.sync_copy(urows, u_hbm.at[pl.ds(base + j * IDXW, IDXW)])

        @pl.loop(0, NNP, step=NBUF)
        def _(g):
            gh = []
            for b in range(NBUF):
                dst = nrows.at[pl.ds(b * IDXW, IDXW)]
                src = cw_hbm.at[ni_v.at[pl.ds((g + b) * IDXW, IDXW)]]
                gh.append(pltpu.async_copy(src, dst, gsem[b]))
            sh = []
            for b in range(NBUF):
                gh[b].wait()
                src = nrows.at[pl.ds(b * IDXW, IDXW)]
                sh.append(pltpu.async_copy(src, sh_ns.at[si_v.at[g + b]],
                                           ssem[b], add=True))
            for b in range(NBUF):
                sh[b].wait()

        @pl.loop(0, NPC)
        def _(p):
            pltpu.sync_copy(
                sh_ns.at[pl.ds(s * BPW + p * IDXW, IDXW)],
                ns_hbm.at[pl.ds(base + p * IDXW, IDXW)],
            )

    return sc_part(target_input, context, ni_flat, sidx2d, target_w, context_w)


def _tc_loss(v, u, ns):
    B, D = v.shape
    TB = 2048

    def body(v_ref, u_ref, ns_ref, o_ref):
        vv = v_ref[...]
        sp = jnp.sum(u_ref[...] * vv, axis=1, keepdims=True)
        sn = jnp.sum(ns_ref[...] * vv, axis=1, keepdims=True)
        ls = jax.nn.log_sigmoid(sp) + jax.nn.log_sigmoid(-sn)
        part = jnp.full((1, 1), -jnp.sum(ls) / B, jnp.float32)

        @pl.when(pl.program_id(0) == 0)
        def _():
            o_ref[...] = jnp.zeros((1, 1), jnp.float32)

        o_ref[...] += part

    out = pl.pallas_call(
        body,
        grid=(B // TB,),
        in_specs=[pl.BlockSpec((TB, D), lambda i: (i, 0))] * 3,
        out_specs=pl.BlockSpec((1, 1), lambda i: (0, 0)),
        out_shape=jax.ShapeDtypeStruct((1, 1), jnp.float32),
    )(v, u, ns)
    return out[0, 0]


def kernel(target_input, context, neg, target_w, context_w):
    B, NEG = neg.shape
    BPW = B // NW

    ni_flat = neg.reshape(-1)
    b = jnp.arange(B, dtype=jnp.int32)
    lidx = ((b // BPW) // NC) * BPW + (b % BPW)
    sidx2d = jnp.repeat(lidx, NEG).reshape(B * NEG // IDXW, IDXW)

    v, u, ns = _sc_gather(target_input, context, ni_flat, sidx2d, target_w, context_w)
    return _tc_loss(v, u, ns)
